# baseline (speedup 1.0000x reference)
import functools

import jax
import jax.numpy as jnp
from jax.experimental import pallas as pl
from jax.experimental.pallas import tpu as pltpu


def _lstm_rec_kernel(
    x_ref,
    zb_ref,
    wx_ref,
    wh_ref,
    hall_ref,
    h_scr,
    c_scr,
    htmp_scr,
    xp_scr,
    *, tt,
):
    tb = pl.program_id(0)
    _, B, E = x_ref.shape
    H = h_scr.shape[1]

    @pl.when(tb == 0)
    def _init():
        h_scr[...] = jnp.zeros_like(h_scr)
        c_scr[...] = jnp.zeros_like(c_scr)

    xp = jnp.dot(x_ref[...].reshape(tt * B, E), wx_ref[...],
                 preferred_element_type=jnp.float32)
    xp_scr[...] = (xp.reshape(tt, B, 4 * H) + zb_ref[...][None]).astype(
        xp_scr.dtype)

    h = h_scr[...]
    c = c_scr[...]
    for s in range(tt):
        gates = xp_scr[s] + jnp.dot(
            h, wh_ref[...], preferred_element_type=jnp.float32)
        ifo = jax.nn.sigmoid(gates[:, :3 * H])
        g = jnp.tanh(gates[:, 3 * H:])
        c = ifo[:, H:2 * H] * c + ifo[:, :H] * g
        h = (ifo[:, 2 * H:] * jnp.tanh(c)).astype(jnp.bfloat16)
        htmp_scr[s] = h
    h_scr[...] = h
    c_scr[...] = c
    hall_ref[...] = jnp.transpose(htmp_scr[...], (1, 0, 2))


def _proj_kernel(h_ref, w_ref, b_ref, out_ref):
    m = pl.program_id(1)
    bm = out_ref.shape[0]
    hblk = h_ref[pl.ds(m * bm, bm), :]
    out_ref[...] = jnp.dot(hblk, w_ref[...],
                           preferred_element_type=jnp.float32) + b_ref[...]


def kernel(x_emb, z, wx, wz, wh, b, wout, bout):
    B, T, E = x_emb.shape
    H = wh.shape[0]
    V = bout.shape[-1]

    tt = 32 if T % 32 == 0 else (16 if T % 16 == 0 else T)
    n_tb = T // tt

    zb = jnp.dot(z, wz) + b
    x_tm = jnp.transpose(x_emb, (1, 0, 2)).astype(jnp.bfloat16)
    wxb = wx.astype(jnp.bfloat16)
    whb = wh.astype(jnp.bfloat16)

    hall = pl.pallas_call(
        functools.partial(_lstm_rec_kernel, tt=tt),
        grid=(n_tb,),
        in_specs=[
            pl.BlockSpec((tt, B, E), lambda tb: (tb, 0, 0)),
            pl.BlockSpec((B, 4 * H), lambda tb: (0, 0)),
            pl.BlockSpec((E, 4 * H), lambda tb: (0, 0)),
            pl.BlockSpec((H, 4 * H), lambda tb: (0, 0)),
        ],
        out_specs=pl.BlockSpec((B, tt, H), lambda tb: (0, tb, 0)),
        out_shape=jax.ShapeDtypeStruct((B, T, H), jnp.bfloat16),
        scratch_shapes=[
            pltpu.VMEM((B, H), jnp.bfloat16),
            pltpu.VMEM((B, H), jnp.float32),
            pltpu.VMEM((tt, B, H), jnp.bfloat16),
            pltpu.VMEM((tt, B, 4 * H), jnp.bfloat16),
        ],
        compiler_params=pltpu.CompilerParams(
            dimension_semantics=("arbitrary",),
        ),
    )(x_tm, zb, wxb, whb)

    M = B * T
    hflat = hall.reshape(M, H)
    bm = 1024 if M % 1024 == 0 else M
    bn = 2048 if V % 2048 == 0 else V
    n_m = M // bm
    n_n = V // bn

    out = pl.pallas_call(
        _proj_kernel,
        grid=(n_n, n_m),
        in_specs=[
            pl.BlockSpec((M, H), lambda n, m: (0, 0)),
            pl.BlockSpec((H, bn), lambda n, m: (0, n)),
            pl.BlockSpec((1, bn), lambda n, m: (0, n)),
        ],
        out_specs=pl.BlockSpec((bm, bn), lambda n, m: (m, n)),
        out_shape=jax.ShapeDtypeStruct((M, V), jnp.float32),
        compiler_params=pltpu.CompilerParams(
            dimension_semantics=("arbitrary", "arbitrary"),
        ),
    )(hflat, wout, bout)
    return out.reshape(B, T, V)


# speedup vs baseline: 1.1195x; 1.1195x over previous
import functools

import jax
import jax.numpy as jnp
from jax.experimental import pallas as pl
from jax.experimental.pallas import tpu as pltpu


def _lstm_rec_kernel(
    x_ref,
    z_ref,
    wz_ref,
    b_ref,
    wx_ref,
    wh_ref,
    hall_ref,
    h_scr,
    c_scr,
    zb_scr,
    htmp_scr,
    xp_scr,
    *, tt,
):
    tb = pl.program_id(0)
    B, _, E = x_ref.shape
    H = h_scr.shape[1]

    @pl.when(tb == 0)
    def _init():
        h_scr[...] = jnp.zeros_like(h_scr)
        c_scr[...] = jnp.zeros_like(c_scr)
        zb_scr[...] = jnp.dot(z_ref[...], wz_ref[...],
                              preferred_element_type=jnp.float32) + b_ref[...]

    xt = jnp.transpose(x_ref[...].astype(jnp.bfloat16), (1, 0, 2))
    xp = jnp.dot(xt.reshape(tt * B, E), wx_ref[...],
                 preferred_element_type=jnp.float32)
    xp_scr[...] = xp.reshape(tt, B, 4 * H) + zb_scr[...][None]

    h = h_scr[...]
    c = c_scr[...]
    for s in range(tt):
        gates = xp_scr[s] + jnp.dot(
            h, wh_ref[...], preferred_element_type=jnp.float32)
        ifo = jax.nn.sigmoid(gates[:, :3 * H])
        g = jnp.tanh(gates[:, 3 * H:])
        c = ifo[:, H:2 * H] * c + ifo[:, :H] * g
        h = (ifo[:, 2 * H:] * jnp.tanh(c)).astype(jnp.bfloat16)
        htmp_scr[s] = h
    h_scr[...] = h
    c_scr[...] = c
    hall_ref[...] = jnp.transpose(htmp_scr[...], (1, 0, 2))


def _proj_kernel(h_ref, w_ref, b_ref, out_ref):
    m = pl.program_id(1)
    bm = out_ref.shape[0]
    hblk = h_ref[pl.ds(m * bm, bm), :]
    out_ref[...] = jnp.dot(hblk, w_ref[...],
                           preferred_element_type=jnp.float32) + b_ref[...]


def kernel(x_emb, z, wx, wz, wh, b, wout, bout):
    B, T, E = x_emb.shape
    Z = z.shape[1]
    H = wh.shape[0]
    V = bout.shape[-1]

    tt = 16 if T % 16 == 0 else T
    n_tb = T // tt

    wxb = wx.astype(jnp.bfloat16)
    whb = wh.astype(jnp.bfloat16)

    hall = pl.pallas_call(
        functools.partial(_lstm_rec_kernel, tt=tt),
        grid=(n_tb,),
        in_specs=[
            pl.BlockSpec((B, tt, E), lambda tb: (0, tb, 0)),
            pl.BlockSpec((B, Z), lambda tb: (0, 0)),
            pl.BlockSpec((Z, 4 * H), lambda tb: (0, 0)),
            pl.BlockSpec((1, 4 * H), lambda tb: (0, 0)),
            pl.BlockSpec((E, 4 * H), lambda tb: (0, 0)),
            pl.BlockSpec((H, 4 * H), lambda tb: (0, 0)),
        ],
        out_specs=pl.BlockSpec((B, tt, H), lambda tb: (0, tb, 0)),
        out_shape=jax.ShapeDtypeStruct((B, T, H), jnp.bfloat16),
        scratch_shapes=[
            pltpu.VMEM((B, H), jnp.bfloat16),
            pltpu.VMEM((B, H), jnp.float32),
            pltpu.VMEM((B, 4 * H), jnp.float32),
            pltpu.VMEM((tt, B, H), jnp.bfloat16),
            pltpu.VMEM((tt, B, 4 * H), jnp.float32),
        ],
        compiler_params=pltpu.CompilerParams(
            dimension_semantics=("arbitrary",),
        ),
    )(x_emb, z, wz, b, wxb, whb)

    M = B * T
    hflat = hall.reshape(M, H)
    bm = 1024 if M % 1024 == 0 else M
    bn = 2048 if V % 2048 == 0 else V
    n_m = M // bm
    n_n = V // bn

    out = pl.pallas_call(
        _proj_kernel,
        grid=(n_n, n_m),
        in_specs=[
            pl.BlockSpec((M, H), lambda n, m: (0, 0)),
            pl.BlockSpec((H, bn), lambda n, m: (0, n)),
            pl.BlockSpec((1, bn), lambda n, m: (0, n)),
        ],
        out_specs=pl.BlockSpec((bm, bn), lambda n, m: (m, n)),
        out_shape=jax.ShapeDtypeStruct((M, V), jnp.float32),
        compiler_params=pltpu.CompilerParams(
            dimension_semantics=("arbitrary", "arbitrary"),
        ),
    )(hflat, wout, bout)
    return out.reshape(B, T, V)
